# baseline (speedup 1.0000x reference)
import functools

import jax
import jax.numpy as jnp
from jax import lax
from jax.experimental import pallas as pl
from jax.experimental.pallas import tpu as pltpu


def _topk_mean_rows(a_abs, k):
    u = lax.bitcast_convert_type(a_abs, jnp.int32)
    t = jnp.zeros((a_abs.shape[0], 1), jnp.int32)
    for bit in range(30, -1, -1):
        cand = t | jnp.int32(1 << bit)
        cnt = jnp.sum((u >= cand).astype(jnp.int32), axis=1, keepdims=True)
        t = jnp.where(cnt >= k, cand, t)
    kth = lax.bitcast_convert_type(t, jnp.float32)
    gt = u > t
    cnt_gt = jnp.sum(gt.astype(jnp.int32), axis=1, keepdims=True)
    sum_gt = jnp.sum(jnp.where(gt, a_abs, jnp.float32(0.0)), axis=1,
                     keepdims=True)
    total = sum_gt + (jnp.float32(k) - cnt_gt.astype(jnp.float32)) * kth
    return total / jnp.float32(k)


def _body(k, bblk, hwblk, nj, x_ref, w_ref, b_ref, o_ref, acc_ref):
    j = pl.program_id(1)
    xb = x_ref[...]
    w = w_ref[...]
    wb = jnp.broadcast_to(w[None, :, :], (bblk, 1, w.shape[1]))
    s = lax.dot_general(
        wb, xb, (((2,), (2,)), ((0,), (0,))),
        preferred_element_type=jnp.float32,
    )[:, 0, :]
    s = s + b_ref[0]
    for jc in range(nj):
        @pl.when(j == jc)
        def _():
            acc_ref[:, jc * hwblk:(jc + 1) * hwblk] = s

    @pl.when(j == nj - 1)
    def _():
        o_ref[...] = _topk_mean_rows(jnp.abs(acc_ref[...]), k)


def kernel(x, W, b):
    B, C, H, Wd = x.shape
    HW = H * Wd
    k = max(int(HW * 0.1), 1)
    bblk = 64
    nj = 32
    hwblk = HW // nj
    xr = x.transpose(0, 2, 3, 1).reshape(B, HW, C)
    wv = W.reshape(1, C)
    out = pl.pallas_call(
        functools.partial(_body, k, bblk, hwblk, nj),
        grid=(B // bblk, nj),
        in_specs=[
            pl.BlockSpec((bblk, hwblk, C), lambda i, j: (i, j, 0)),
            pl.BlockSpec((1, C), lambda i, j: (0, 0)),
            pl.BlockSpec(memory_space=pltpu.SMEM),
        ],
        out_specs=pl.BlockSpec((bblk, 1), lambda i, j: (i, 0)),
        out_shape=jax.ShapeDtypeStruct((B, 1), jnp.float32),
        scratch_shapes=[pltpu.VMEM((bblk, HW), jnp.float32)],
    )(xr, wv, b)
    return out


# speedup vs baseline: 1.0683x; 1.0683x over previous
import functools

import jax
import jax.numpy as jnp
from jax import lax
from jax.experimental import pallas as pl
from jax.experimental.pallas import tpu as pltpu


def _topk_mean_rows(a_abs, k):
    u = lax.bitcast_convert_type(a_abs, jnp.int32)
    t = jnp.zeros((a_abs.shape[0], 1), jnp.int32)
    for bit in range(30, -1, -1):
        cand = t | jnp.int32(1 << bit)
        cnt = jnp.sum((u >= cand).astype(jnp.int32), axis=1, keepdims=True)
        t = jnp.where(cnt >= k, cand, t)
    kth = lax.bitcast_convert_type(t, jnp.float32)
    gt = u > t
    cnt_gt = jnp.sum(gt.astype(jnp.int32), axis=1, keepdims=True)
    sum_gt = jnp.sum(jnp.where(gt, a_abs, jnp.float32(0.0)), axis=1,
                     keepdims=True)
    total = sum_gt + (jnp.float32(k) - cnt_gt.astype(jnp.float32)) * kth
    return total / jnp.float32(k)


def _body(k, bblk, hwblk, nj, x_ref, w_ref, b_ref, o_ref, acc_ref):
    j = pl.program_id(1)
    xb = x_ref[...]
    w = w_ref[...]
    wb = jnp.broadcast_to(w[None, :, :], (bblk, 1, w.shape[1]))
    s = lax.dot_general(
        wb, xb, (((2,), (2,)), ((0,), (0,))),
        preferred_element_type=jnp.float32,
    )[:, 0, :]
    s = s + b_ref[0]
    for jc in range(nj):
        @pl.when(j == jc)
        def _():
            acc_ref[:, jc * hwblk:(jc + 1) * hwblk] = s

    @pl.when(j == nj - 1)
    def _():
        o_ref[...] = _topk_mean_rows(jnp.abs(acc_ref[...]), k)


def kernel(x, W, b):
    B, C, H, Wd = x.shape
    HW = H * Wd
    k = max(int(HW * 0.1), 1)
    bblk = 64
    nj = 16
    hwblk = HW // nj
    xr = x.transpose(0, 2, 3, 1).reshape(B, HW, C)
    wv = W.reshape(1, C)
    out = pl.pallas_call(
        functools.partial(_body, k, bblk, hwblk, nj),
        grid=(B // bblk, nj),
        in_specs=[
            pl.BlockSpec((bblk, hwblk, C), lambda i, j: (i, j, 0)),
            pl.BlockSpec((1, C), lambda i, j: (0, 0)),
            pl.BlockSpec(memory_space=pltpu.SMEM),
        ],
        out_specs=pl.BlockSpec((bblk, 1), lambda i, j: (i, 0)),
        out_shape=jax.ShapeDtypeStruct((B, 1), jnp.float32),
        scratch_shapes=[pltpu.VMEM((bblk, HW), jnp.float32)],
    )(xr, wv, b)
    return out


# speedup vs baseline: 1.1202x; 1.0485x over previous
import functools

import jax
import jax.numpy as jnp
from jax import lax
from jax.experimental import pallas as pl
from jax.experimental.pallas import tpu as pltpu


def _topk_mean_rows(a_abs, k):
    u = lax.bitcast_convert_type(a_abs, jnp.int32)
    t = jnp.zeros((a_abs.shape[0], 1), jnp.int32)

    def _cnt(cand):
        return jnp.sum((u >= cand).astype(jnp.int32), axis=1, keepdims=True)

    bits = list(range(30, -1, -1))
    for i in range(0, 30, 2):
        b1 = jnp.int32(1 << bits[i])
        b2 = jnp.int32(1 << bits[i + 1])
        c1 = t | b1
        c2a = c1 | b2
        c2b = t | b2
        n1, n2a, n2b = _cnt(c1), _cnt(c2a), _cnt(c2b)
        t = jnp.where(n1 >= k,
                      jnp.where(n2a >= k, c2a, c1),
                      jnp.where(n2b >= k, c2b, t))
    cand = t | jnp.int32(1)
    t = jnp.where(_cnt(cand) >= k, cand, t)
    kth = lax.bitcast_convert_type(t, jnp.float32)
    gt = u > t
    cnt_gt = jnp.sum(gt.astype(jnp.int32), axis=1, keepdims=True)
    sum_gt = jnp.sum(jnp.where(gt, a_abs, jnp.float32(0.0)), axis=1,
                     keepdims=True)
    total = sum_gt + (jnp.float32(k) - cnt_gt.astype(jnp.float32)) * kth
    return total / jnp.float32(k)


def _body(k, bblk, hwblk, nj, x_ref, w_ref, b_ref, o_ref, acc_ref):
    j = pl.program_id(1)
    xb = x_ref[...]
    w = w_ref[...]
    wb = jnp.broadcast_to(w[None, :, :], (bblk, 1, w.shape[1]))
    s = lax.dot_general(
        wb, xb, (((2,), (2,)), ((0,), (0,))),
        preferred_element_type=jnp.float32,
    )[:, 0, :]
    s = s + b_ref[0]
    for jc in range(nj):
        @pl.when(j == jc)
        def _():
            acc_ref[:, jc * hwblk:(jc + 1) * hwblk] = s

    @pl.when(j == nj - 1)
    def _():
        o_ref[...] = _topk_mean_rows(jnp.abs(acc_ref[...]), k)


def kernel(x, W, b):
    B, C, H, Wd = x.shape
    HW = H * Wd
    k = max(int(HW * 0.1), 1)
    bblk = 64
    nj = 16
    hwblk = HW // nj
    xr = x.transpose(0, 2, 3, 1).reshape(B, HW, C)
    wv = W.reshape(1, C)
    out = pl.pallas_call(
        functools.partial(_body, k, bblk, hwblk, nj),
        grid=(B // bblk, nj),
        in_specs=[
            pl.BlockSpec((bblk, hwblk, C), lambda i, j: (i, j, 0)),
            pl.BlockSpec((1, C), lambda i, j: (0, 0)),
            pl.BlockSpec(memory_space=pltpu.SMEM),
        ],
        out_specs=pl.BlockSpec((bblk, 1), lambda i, j: (i, 0)),
        out_shape=jax.ShapeDtypeStruct((B, 1), jnp.float32),
        scratch_shapes=[pltpu.VMEM((bblk, HW), jnp.float32)],
    )(xr, wv, b)
    return out
